# Initial kernel scaffold; baseline (speedup 1.0000x reference)
import jax
import jax.numpy as jnp
from jax.experimental import pallas as pl


def kernel(x, edge_index, batch, W1, b1, W2, b2, Wm1, bm1, Wm2, bm2):
    raise NotImplementedError("write your pallas kernel here")


# speedup vs baseline: 11.7400x; 11.7400x over previous
import functools

import jax
import jax.numpy as jnp
from jax import lax
from jax.experimental import pallas as pl
from jax.experimental.pallas import tpu as pltpu
import jax.experimental.pallas.tpu_sc as plsc

N = 10000
NPAD = 10240
F = 128
G = 64
NC = 2
NS = 16
NW = NC * NS
CHUNK = 128
ROWS_PER_TILE = NPAD // NS
BN = 1024
NBLK = NPAD // BN


def _deg_body(epw, dst_hbm, zeros_hbm, out_hbm, idx_v, ones_v, deg_sp):
    c = lax.axis_index("c")
    s = lax.axis_index("s")
    w = c * NS + s
    for i in range(CHUNK // 16):
        ones_v[pl.ds(i * 16, 16)] = jnp.ones((16,), jnp.float32)
    pltpu.sync_copy(zeros_hbm.at[pl.ds(s * ROWS_PER_TILE, ROWS_PER_TILE)],
                    deg_sp.at[pl.ds(s * ROWS_PER_TILE, ROWS_PER_TILE)])
    plsc.subcore_barrier()

    def body(j, carry):
        pltpu.sync_copy(dst_hbm.at[pl.ds(w * epw + j * CHUNK, CHUNK)], idx_v)
        pltpu.sync_copy(ones_v, deg_sp.at[idx_v], add=True)
        return carry

    lax.fori_loop(0, epw // CHUNK, body, 0)
    plsc.subcore_barrier()
    pltpu.sync_copy(deg_sp.at[pl.ds(s * ROWS_PER_TILE, ROWS_PER_TILE)],
                    out_hbm.at[c, pl.ds(s * ROWS_PER_TILE, ROWS_PER_TILE)])


def _prop_body(epw, hs_hbm, src_hbm, dst_hbm, zeros_hbm, out_hbm,
               srcs_v, didx_v, rows_v, agg_sp, sem):
    c = lax.axis_index("c")
    s = lax.axis_index("s")
    w = c * NS + s
    pltpu.sync_copy(zeros_hbm.at[pl.ds(s * ROWS_PER_TILE, ROWS_PER_TILE)],
                    agg_sp.at[pl.ds(s * ROWS_PER_TILE, ROWS_PER_TILE)])
    pltpu.sync_copy(src_hbm.at[pl.ds(w * epw, epw)], srcs_v)
    plsc.subcore_barrier()

    def body(j, carry):
        pltpu.async_copy(
            hs_hbm.at[srcs_v.at[pl.ds(j * CHUNK, CHUNK)]], rows_v, sem).wait()
        pltpu.sync_copy(dst_hbm.at[pl.ds(w * epw + j * CHUNK, CHUNK)], didx_v)
        pltpu.sync_copy(rows_v, agg_sp.at[didx_v], add=True)
        return carry

    lax.fori_loop(0, epw // CHUNK, body, 0)
    plsc.subcore_barrier()
    pltpu.sync_copy(agg_sp.at[pl.ds(s * ROWS_PER_TILE, ROWS_PER_TILE)],
                    out_hbm.at[c, pl.ds(s * ROWS_PER_TILE, ROWS_PER_TILE)])


def _make_deg_call(epw):
    return pl.kernel(
        functools.partial(_deg_body, epw),
        out_type=jax.ShapeDtypeStruct((NC, NPAD), jnp.float32),
        mesh=plsc.VectorSubcoreMesh(core_axis_name="c", subcore_axis_name="s"),
        scratch_types=[
            pltpu.VMEM((CHUNK,), jnp.int32),
            pltpu.VMEM((CHUNK,), jnp.float32),
            pltpu.VMEM_SHARED((NPAD,), jnp.float32),
        ],
    )


def _make_prop_call(epw):
    return pl.kernel(
        functools.partial(_prop_body, epw),
        out_type=jax.ShapeDtypeStruct((NC, NPAD, F), jnp.float32),
        mesh=plsc.VectorSubcoreMesh(core_axis_name="c", subcore_axis_name="s"),
        scratch_types=[
            pltpu.VMEM((epw,), jnp.int32),
            pltpu.VMEM((CHUNK,), jnp.int32),
            pltpu.VMEM((CHUNK, F), jnp.float32),
            pltpu.VMEM_SHARED((NPAD, F), jnp.float32),
            pltpu.SemaphoreType.DMA,
        ],
    )


def _dinv(degp_ref):
    deg = 1.0 + degp_ref[0] + degp_ref[1]
    return lax.rsqrt(deg)


def _mm_scale_body(degp_ref, x_ref, w_ref, o_ref):
    dinv = _dinv(degp_ref)
    h = jnp.dot(x_ref[...], w_ref[...], preferred_element_type=jnp.float32)
    o_ref[...] = h * dinv[:, None]


def _layer_mid_body(degp_ref, agg_ref, hs_ref, b_ref, w_ref, o_ref):
    dinv = _dinv(degp_ref)
    aggsum = agg_ref[0] + agg_ref[1] + hs_ref[...]
    out1 = jnp.maximum(aggsum * dinv[:, None] + b_ref[...], 0.0)
    h2 = jnp.dot(out1, w_ref[...], preferred_element_type=jnp.float32)
    o_ref[...] = h2 * dinv[:, None]


def _final_body(degp_ref, agg_ref, hs_ref, b_ref, batch_ref,
                wm1_ref, bm1_ref, wm2_ref, bm2_ref, o_ref,
                pooled_acc, counts_acc):
    i = pl.program_id(0)
    dinv = _dinv(degp_ref)
    aggsum = agg_ref[0] + agg_ref[1] + hs_ref[...]
    out2 = jnp.maximum(aggsum * dinv[:, None] + b_ref[...], 0.0)
    bt = batch_ref[0]
    onehot = (bt[:, None] == lax.broadcasted_iota(jnp.int32, (BN, G), 1)
              ).astype(jnp.float32)

    @pl.when(i == 0)
    def _():
        pooled_acc[...] = jnp.zeros_like(pooled_acc)
        counts_acc[...] = jnp.zeros_like(counts_acc)

    pooled_acc[...] += lax.dot_general(
        onehot, out2, (((0,), (0,)), ((), ())),
        preferred_element_type=jnp.float32)
    counts_acc[...] += jnp.sum(onehot, axis=0)[None, :]

    @pl.when(i == NBLK - 1)
    def _():
        counts = jnp.maximum(counts_acc[0], 1.0)
        pooled = pooled_acc[...] / counts[:, None]
        z = jnp.maximum(
            jnp.dot(pooled, wm1_ref[...], preferred_element_type=jnp.float32)
            + bm1_ref[...], 0.0)
        o_ref[...] = jnp.dot(
            z, wm2_ref[...], preferred_element_type=jnp.float32) + bm2_ref[...]


_degp_spec = pl.BlockSpec((NC, BN), lambda i: (0, i))
_row_spec = pl.BlockSpec((BN, F), lambda i: (i, 0))
_agg_spec = pl.BlockSpec((NC, BN, F), lambda i: (0, i, 0))
_full = lambda shape: pl.BlockSpec(shape, lambda i: tuple(0 for _ in shape))


def _mm_scale(degp, x, w):
    return pl.pallas_call(
        _mm_scale_body,
        grid=(NBLK,),
        in_specs=[_degp_spec, _row_spec, _full((F, F))],
        out_specs=_row_spec,
        out_shape=jax.ShapeDtypeStruct((NPAD, F), jnp.float32),
    )(degp, x, w)


def _layer_mid(degp, agg, hs, b, w):
    return pl.pallas_call(
        _layer_mid_body,
        grid=(NBLK,),
        in_specs=[_degp_spec, _agg_spec, _row_spec, _full((1, F)),
                  _full((F, F))],
        out_specs=_row_spec,
        out_shape=jax.ShapeDtypeStruct((NPAD, F), jnp.float32),
    )(degp, agg, hs, b, w)


def _final(degp, agg, hs, b, batch2d, wm1, bm1, wm2, bm2):
    return pl.pallas_call(
        _final_body,
        grid=(NBLK,),
        in_specs=[_degp_spec, _agg_spec, _row_spec, _full((1, F)),
                  pl.BlockSpec((1, BN), lambda i: (0, i)),
                  _full((F, 2 * F)), _full((1, 2 * F)),
                  _full((2 * F, F)), _full((1, F))],
        out_specs=_full((G, F)),
        out_shape=jax.ShapeDtypeStruct((G, F), jnp.float32),
        scratch_shapes=[pltpu.VMEM((G, F), jnp.float32),
                        pltpu.VMEM((1, G), jnp.float32)],
    )(degp, agg, hs, b, batch2d, wm1, bm1, wm2, bm2)


def kernel(x, edge_index, batch, W1, b1, W2, b2, Wm1, bm1, Wm2, bm2):
    E = edge_index.shape[1]
    epw = -(-E // (NW * CHUNK)) * CHUNK
    EPAD = epw * NW

    src = jnp.concatenate(
        [edge_index[0], jnp.zeros((EPAD - E,), jnp.int32)])
    dst = jnp.concatenate(
        [edge_index[1], jnp.full((EPAD - E,), N, jnp.int32)])
    xp = jnp.concatenate([x, jnp.zeros((NPAD - N, F), x.dtype)])
    batch2d = jnp.concatenate(
        [batch, jnp.full((NPAD - N,), G, batch.dtype)]).reshape(1, NPAD)
    zeros1 = jnp.zeros((NPAD,), jnp.float32)
    zeros2 = jnp.zeros((NPAD, F), jnp.float32)
    b1r = b1.reshape(1, F)
    b2r = b2.reshape(1, F)
    bm1r = bm1.reshape(1, 2 * F)
    K2 = Wm2.shape[1]
    wm2p = jnp.concatenate([Wm2, jnp.zeros((2 * F, F - K2), Wm2.dtype)], axis=1)
    bm2p = jnp.concatenate([bm2, jnp.zeros((F - K2,), bm2.dtype)]).reshape(1, F)

    degp = _make_deg_call(epw)(dst, zeros1)
    hs1 = _mm_scale(degp, xp, W1)
    prop = _make_prop_call(epw)
    agg1 = prop(hs1, src, dst, zeros2)
    hs2 = _layer_mid(degp, agg1, hs1, b1r, W2)
    agg2 = prop(hs2, src, dst, zeros2)
    out = _final(degp, agg2, hs2, b2r, batch2d, Wm1, bm1r, wm2p, bm2p)
    return out[:, :K2].reshape(G, K2 // 2, 2)
